# initial kernel scaffold (unmeasured)
import jax
import jax.numpy as jnp
from jax import lax
from jax.experimental import pallas as pl
from jax.experimental.pallas import tpu as pltpu

T_SHARD = 256
D = 512
F = 1024
E_LOCAL = 2


def kernel(x, router, W1, W2):
    def body(x_ref, r_ref, w1_ref, w2_ref, out_ref,
             xr_ref, rr_ref, pmine_ref, psend_ref, precv_ref,
             send_sems, recv_sems):
        my_x = lax.axis_index("x")
        my_y = lax.axis_index("y")
        nbr = (1 - my_x, my_y)

        barrier_sem = pltpu.get_barrier_semaphore()
        pl.semaphore_signal(barrier_sem, inc=1, device_id=nbr,
                            device_id_type=pl.DeviceIdType.MESH)
        pl.semaphore_wait(barrier_sem, 1)

        rdma_x = pltpu.make_async_remote_copy(
            src_ref=x_ref, dst_ref=xr_ref,
            send_sem=send_sems.at[0], recv_sem=recv_sems.at[0],
            device_id=nbr, device_id_type=pl.DeviceIdType.MESH)
        rdma_x.start()
        rdma_r = pltpu.make_async_remote_copy(
            src_ref=r_ref, dst_ref=rr_ref,
            send_sem=send_sems.at[1], recv_sem=recv_sems.at[1],
            device_id=nbr, device_id_type=pl.DeviceIdType.MESH)
        rdma_r.start()
        rdma_r.wait()
        rdma_x.wait()

        def expert_contrib(xv, g):
            raise NotImplementedError

        def compute(mx):
            for mine in (True, False):
                xv = x_ref[:, :] if mine else xr_ref[:, :]
                gl = jnp.dot(xv, r_ref[:, :],
                             preferred_element_type=jnp.float32)
                gr = jnp.dot(xv, rr_ref[:, :],
                             preferred_element_type=jnp.float32)
                if mx == 0:
                    g = jnp.concatenate([gl, gr], axis=1)
                else:
                    g = jnp.concatenate([gr, gl], axis=1)
                m1 = jnp.max(g, axis=1, keepdims=True)
                eq1 = (g == m1).astype(jnp.float32)
                top1 = eq1 * (jnp.cumsum(eq1, axis=1) == 1.0)
                g2 = jnp.where(top1 > 0.0, -1e30, g)
                m2 = jnp.max(g2, axis=1, keepdims=True)
                eq2 = (g2 == m2).astype(jnp.float32)
                top2 = eq2 * (jnp.cumsum(eq2, axis=1) == 1.0)
                b = jnp.exp(m2 - m1)
                wa = 1.0 / (1.0 + b)
                wb = b / (1.0 + b)
                we = top1 * wa + top2 * wb

                c0 = we[:, 2 * mx:2 * mx + 1]
                c1 = we[:, 2 * mx + 1:2 * mx + 2]
                h0 = jnp.maximum(jnp.dot(xv, w1_ref[0, :, :],
                                         preferred_element_type=jnp.float32),
                                 0.0)
                y0 = jnp.dot(h0, w2_ref[0, :, :],
                             preferred_element_type=jnp.float32)
                h1 = jnp.maximum(jnp.dot(xv, w1_ref[1, :, :],
                                         preferred_element_type=jnp.float32),
                                 0.0)
                y1 = jnp.dot(h1, w2_ref[1, :, :],
                             preferred_element_type=jnp.float32)
                p = y0 * c0 + y1 * c1
                if mine:
                    pmine_ref[:, :] = p
                else:
                    psend_ref[:, :] = p

        @pl.when(my_x == 0)
        def _():
            compute(0)

        @pl.when(my_x == 1)
        def _():
            compute(1)

        rdma_p = pltpu.make_async_remote_copy(
            src_ref=psend_ref, dst_ref=precv_ref,
            send_sem=send_sems.at[2], recv_sem=recv_sems.at[2],
            device_id=nbr, device_id_type=pl.DeviceIdType.MESH)
        rdma_p.start()
        rdma_p.wait()

        out_ref[:, :] = pmine_ref[:, :] + precv_ref[:, :]

    return pl.pallas_call(
        body,
        out_shape=jax.ShapeDtypeStruct((T_SHARD, D), jnp.float32),
        in_specs=[pl.BlockSpec(memory_space=pltpu.VMEM)] * 4,
        out_specs=pl.BlockSpec(memory_space=pltpu.VMEM),
        scratch_shapes=[
            pltpu.VMEM((T_SHARD, D), jnp.float32),
            pltpu.VMEM((D, E_LOCAL), jnp.float32),
            pltpu.VMEM((T_SHARD, D), jnp.float32),
            pltpu.VMEM((T_SHARD, D), jnp.float32),
            pltpu.VMEM((T_SHARD, D), jnp.float32),
            pltpu.SemaphoreType.DMA((3,)),
            pltpu.SemaphoreType.DMA((3,)),
        ],
        compiler_params=pltpu.CompilerParams(collective_id=0),
    )(x, router, W1, W2)


# baseline (device time: 30242 ns/iter reference)
import jax
import jax.numpy as jnp
from jax import lax
from jax.experimental import pallas as pl
from jax.experimental.pallas import tpu as pltpu

T_SHARD = 256
D = 512
F = 1024
E_LOCAL = 2


def kernel(x, router, W1, W2):
    def body(x_ref, r_ref, w1_ref, w2_ref, out_ref,
             xr_ref, rr_ref, pmine_ref, psend_ref, precv_ref,
             send_sems, recv_sems):
        my_x = lax.axis_index("x")
        my_y = lax.axis_index("y")
        nbr = (1 - my_x, my_y)

        barrier_sem = pltpu.get_barrier_semaphore()
        pl.semaphore_signal(barrier_sem, inc=1, device_id=nbr,
                            device_id_type=pl.DeviceIdType.MESH)
        pl.semaphore_wait(barrier_sem, 1)

        rdma_x = pltpu.make_async_remote_copy(
            src_ref=x_ref, dst_ref=xr_ref,
            send_sem=send_sems.at[0], recv_sem=recv_sems.at[0],
            device_id=nbr, device_id_type=pl.DeviceIdType.MESH)
        rdma_x.start()
        rdma_r = pltpu.make_async_remote_copy(
            src_ref=r_ref, dst_ref=rr_ref,
            send_sem=send_sems.at[1], recv_sem=recv_sems.at[1],
            device_id=nbr, device_id_type=pl.DeviceIdType.MESH)
        rdma_r.start()
        rdma_r.wait()
        rdma_x.wait()

        def compute(mx):
            for mine in (True, False):
                xv = x_ref[:, :] if mine else xr_ref[:, :]
                gl = jnp.dot(xv, r_ref[:, :],
                             preferred_element_type=jnp.float32)
                gr = jnp.dot(xv, rr_ref[:, :],
                             preferred_element_type=jnp.float32)
                if mx == 0:
                    g = jnp.concatenate([gl, gr], axis=1)
                else:
                    g = jnp.concatenate([gr, gl], axis=1)
                def first_hit(eq):
                    z = jnp.zeros_like(eq[:, 0:1])
                    ps = jnp.concatenate(
                        [z,
                         eq[:, 0:1],
                         eq[:, 0:1] + eq[:, 1:2],
                         eq[:, 0:1] + eq[:, 1:2] + eq[:, 2:3]],
                        axis=1)
                    return eq * (ps == 0.0)

                m1 = jnp.max(g, axis=1, keepdims=True)
                top1 = first_hit((g == m1).astype(jnp.float32))
                g2 = jnp.where(top1 > 0.0, -1e30, g)
                m2 = jnp.max(g2, axis=1, keepdims=True)
                top2 = first_hit((g2 == m2).astype(jnp.float32))
                b = jnp.exp(m2 - m1)
                wa = 1.0 / (1.0 + b)
                wb = b / (1.0 + b)
                we = top1 * wa + top2 * wb

                c0 = we[:, 2 * mx:2 * mx + 1]
                c1 = we[:, 2 * mx + 1:2 * mx + 2]
                h0 = jnp.maximum(jnp.dot(xv, w1_ref[0, :, :],
                                         preferred_element_type=jnp.float32),
                                 0.0)
                y0 = jnp.dot(h0, w2_ref[0, :, :],
                             preferred_element_type=jnp.float32)
                h1 = jnp.maximum(jnp.dot(xv, w1_ref[1, :, :],
                                         preferred_element_type=jnp.float32),
                                 0.0)
                y1 = jnp.dot(h1, w2_ref[1, :, :],
                             preferred_element_type=jnp.float32)
                p = y0 * c0 + y1 * c1
                if mine:
                    pmine_ref[:, :] = p
                else:
                    psend_ref[:, :] = p

        @pl.when(my_x == 0)
        def _():
            compute(0)

        @pl.when(my_x == 1)
        def _():
            compute(1)

        rdma_p = pltpu.make_async_remote_copy(
            src_ref=psend_ref, dst_ref=precv_ref,
            send_sem=send_sems.at[2], recv_sem=recv_sems.at[2],
            device_id=nbr, device_id_type=pl.DeviceIdType.MESH)
        rdma_p.start()
        rdma_p.wait()

        out_ref[:, :] = pmine_ref[:, :] + precv_ref[:, :]

    return pl.pallas_call(
        body,
        out_shape=jax.ShapeDtypeStruct((T_SHARD, D), jnp.float32),
        in_specs=[pl.BlockSpec(memory_space=pltpu.VMEM)] * 4,
        out_specs=pl.BlockSpec(memory_space=pltpu.VMEM),
        scratch_shapes=[
            pltpu.VMEM((T_SHARD, D), jnp.float32),
            pltpu.VMEM((D, E_LOCAL), jnp.float32),
            pltpu.VMEM((T_SHARD, D), jnp.float32),
            pltpu.VMEM((T_SHARD, D), jnp.float32),
            pltpu.VMEM((T_SHARD, D), jnp.float32),
            pltpu.SemaphoreType.DMA((3,)),
            pltpu.SemaphoreType.DMA((3,)),
        ],
        compiler_params=pltpu.CompilerParams(collective_id=0),
    )(x, router, W1, W2)


# device time: 30236 ns/iter; 1.0002x vs baseline; 1.0002x over previous
import jax
import jax.numpy as jnp
from jax import lax
from jax.experimental import pallas as pl
from jax.experimental.pallas import tpu as pltpu

T_SHARD = 256
D = 512
F = 1024
E_LOCAL = 2


def kernel(x, router, W1, W2):
    def body(x_ref, r_ref, w1_ref, w2_ref, out_ref,
             xr_ref, rr_ref, pmine_ref, psend_ref, precv_ref,
             send_sems, recv_sems):
        my_x = lax.axis_index("x")
        my_y = lax.axis_index("y")
        nbr = (1 - my_x, my_y)

        barrier_sem = pltpu.get_barrier_semaphore()
        pl.semaphore_signal(barrier_sem, inc=1, device_id=nbr,
                            device_id_type=pl.DeviceIdType.MESH)
        pl.semaphore_wait(barrier_sem, 1)

        rdma_x = pltpu.make_async_remote_copy(
            src_ref=x_ref, dst_ref=xr_ref,
            send_sem=send_sems.at[0], recv_sem=recv_sems.at[0],
            device_id=nbr, device_id_type=pl.DeviceIdType.MESH)
        rdma_x.start()
        rdma_r = pltpu.make_async_remote_copy(
            src_ref=r_ref, dst_ref=rr_ref,
            send_sem=send_sems.at[1], recv_sem=recv_sems.at[1],
            device_id=nbr, device_id_type=pl.DeviceIdType.MESH)
        rdma_r.start()
        rdma_r.wait()
        rdma_x.wait()

        def compute(mx):
            for mine in (True, False):
                xv = x_ref[:, :] if mine else xr_ref[:, :]
                gl = jnp.dot(xv, r_ref[:, :],
                             preferred_element_type=jnp.float32)
                gr = jnp.dot(xv, rr_ref[:, :],
                             preferred_element_type=jnp.float32)
                if mx == 0:
                    g = jnp.concatenate([gl, gr], axis=1)
                else:
                    g = jnp.concatenate([gr, gl], axis=1)
                def first_hit(eq):
                    z = jnp.zeros_like(eq[:, 0:1])
                    ps = jnp.concatenate(
                        [z,
                         eq[:, 0:1],
                         eq[:, 0:1] + eq[:, 1:2],
                         eq[:, 0:1] + eq[:, 1:2] + eq[:, 2:3]],
                        axis=1)
                    return eq * (ps == 0.0)

                m1 = jnp.max(g, axis=1, keepdims=True)
                top1 = first_hit((g == m1).astype(jnp.float32))
                g2 = jnp.where(top1 > 0.0, -1e30, g)
                m2 = jnp.max(g2, axis=1, keepdims=True)
                top2 = first_hit((g2 == m2).astype(jnp.float32))
                b = jnp.exp(m2 - m1)
                wa = 1.0 / (1.0 + b)
                wb = b / (1.0 + b)
                we = top1 * wa + top2 * wb

                c0 = we[:, 2 * mx:2 * mx + 1]
                c1 = we[:, 2 * mx + 1:2 * mx + 2]
                xb = xv.astype(jnp.bfloat16)
                w10 = w1_ref[0, :, :].astype(jnp.bfloat16)
                w11 = w1_ref[1, :, :].astype(jnp.bfloat16)
                w20 = w2_ref[0, :, :].astype(jnp.bfloat16)
                w21 = w2_ref[1, :, :].astype(jnp.bfloat16)
                h0 = jnp.maximum(jnp.dot(xb, w10,
                                         preferred_element_type=jnp.float32),
                                 0.0).astype(jnp.bfloat16)
                y0 = jnp.dot(h0, w20,
                             preferred_element_type=jnp.float32)
                h1 = jnp.maximum(jnp.dot(xb, w11,
                                         preferred_element_type=jnp.float32),
                                 0.0).astype(jnp.bfloat16)
                y1 = jnp.dot(h1, w21,
                             preferred_element_type=jnp.float32)
                p = y0 * c0 + y1 * c1
                if mine:
                    pmine_ref[:, :] = p
                else:
                    psend_ref[:, :] = p

        @pl.when(my_x == 0)
        def _():
            compute(0)

        @pl.when(my_x == 1)
        def _():
            compute(1)

        rdma_p = pltpu.make_async_remote_copy(
            src_ref=psend_ref, dst_ref=precv_ref,
            send_sem=send_sems.at[2], recv_sem=recv_sems.at[2],
            device_id=nbr, device_id_type=pl.DeviceIdType.MESH)
        rdma_p.start()
        rdma_p.wait()

        out_ref[:, :] = pmine_ref[:, :] + precv_ref[:, :]

    return pl.pallas_call(
        body,
        out_shape=jax.ShapeDtypeStruct((T_SHARD, D), jnp.float32),
        in_specs=[pl.BlockSpec(memory_space=pltpu.VMEM)] * 4,
        out_specs=pl.BlockSpec(memory_space=pltpu.VMEM),
        scratch_shapes=[
            pltpu.VMEM((T_SHARD, D), jnp.float32),
            pltpu.VMEM((D, E_LOCAL), jnp.float32),
            pltpu.VMEM((T_SHARD, D), jnp.float32),
            pltpu.VMEM((T_SHARD, D), jnp.float32),
            pltpu.VMEM((T_SHARD, D), jnp.float32),
            pltpu.SemaphoreType.DMA((3,)),
            pltpu.SemaphoreType.DMA((3,)),
        ],
        compiler_params=pltpu.CompilerParams(collective_id=0),
    )(x, router, W1, W2)


# device time: 19756 ns/iter; 1.5308x vs baseline; 1.5305x over previous
import jax
import jax.numpy as jnp
from jax import lax
from jax.experimental import pallas as pl
from jax.experimental.pallas import tpu as pltpu

T_SHARD = 256
D = 512
F = 1024
E_LOCAL = 2

_HBM = pltpu.MemorySpace.HBM


def kernel(x, router, W1, W2):
    def body(x_hbm, r_hbm, w1_hbm, w2_hbm, out_hbm,
             xv_ref, r_ref, w1_ref, w2_ref, xsend_ref, xrb_ref, rr_ref,
             csend_ref, crecv_ref, pmine_ref, psend_ref, precv_ref,
             send_sems, recv_sems, csems):
        my_x = lax.axis_index("x")
        my_y = lax.axis_index("y")
        nbr = (1 - my_x, my_y)

        xcp = pltpu.make_async_copy(x_hbm, xv_ref, csems.at[0])
        xcp.start()
        rcp = pltpu.make_async_copy(r_hbm, r_ref, csems.at[1])
        rcp.start()
        w1cp = pltpu.make_async_copy(w1_hbm, w1_ref, csems.at[2])
        w1cp.start()
        w2cp = pltpu.make_async_copy(w2_hbm, w2_ref, csems.at[3])
        w2cp.start()

        xcp.wait()
        xsend_ref[:, :] = xv_ref[:, :].astype(jnp.bfloat16)

        barrier_sem = pltpu.get_barrier_semaphore()
        pl.semaphore_signal(barrier_sem, inc=1, device_id=nbr,
                            device_id_type=pl.DeviceIdType.MESH)
        pl.semaphore_wait(barrier_sem, 1)

        rcp.wait()
        rdma_r = pltpu.make_async_remote_copy(
            src_ref=r_ref, dst_ref=rr_ref,
            send_sem=send_sems.at[0], recv_sem=recv_sems.at[0],
            device_id=nbr, device_id_type=pl.DeviceIdType.MESH)
        rdma_r.start()
        rdma_x = pltpu.make_async_remote_copy(
            src_ref=xsend_ref, dst_ref=xrb_ref,
            send_sem=send_sems.at[1], recv_sem=recv_sems.at[1],
            device_id=nbr, device_id_type=pl.DeviceIdType.MESH)
        rdma_x.start()

        def col(we, j):
            eidx = lax.broadcasted_iota(jnp.int32, we.shape, 1)
            return jnp.sum(jnp.where(eidx == j, we, 0.0),
                           axis=1, keepdims=True)

        def weights(xv):
            rfull = jnp.where(
                my_x == 0,
                jnp.concatenate([r_ref[:, :], rr_ref[:, :]], axis=1),
                jnp.concatenate([rr_ref[:, :], r_ref[:, :]], axis=1))
            g = jnp.dot(xv, rfull, preferred_element_type=jnp.float32)

            def first_hit(eq):
                z = jnp.zeros_like(eq[:, 0:1])
                ps = jnp.concatenate(
                    [z,
                     eq[:, 0:1],
                     eq[:, 0:1] + eq[:, 1:2],
                     eq[:, 0:1] + eq[:, 1:2] + eq[:, 2:3]],
                    axis=1)
                return eq * (ps == 0.0)

            m1 = jnp.max(g, axis=1, keepdims=True)
            top1 = first_hit((g == m1).astype(jnp.float32))
            g2 = jnp.where(top1 > 0.0, -1e30, g)
            m2 = jnp.max(g2, axis=1, keepdims=True)
            top2 = first_hit((g2 == m2).astype(jnp.float32))
            b = jnp.exp(m2 - m1)
            return top1 * (1.0 / (1.0 + b)) + top2 * (b / (1.0 + b))

        def ffn(xv):
            h0 = jnp.maximum(jnp.dot(xv, w1_ref[0, :, :],
                                     preferred_element_type=jnp.float32), 0.0)
            y0 = jnp.dot(h0, w2_ref[0, :, :],
                         preferred_element_type=jnp.float32)
            h1 = jnp.maximum(jnp.dot(xv, w1_ref[1, :, :],
                                     preferred_element_type=jnp.float32), 0.0)
            y1 = jnp.dot(h1, w2_ref[1, :, :],
                         preferred_element_type=jnp.float32)
            return y0, y1

        rdma_r.wait_recv()
        we = weights(xv_ref[:, :])
        csend_ref[:, 0:1] = col(we, 2 * (1 - my_x))
        csend_ref[:, 1:2] = col(we, 2 * (1 - my_x) + 1)
        rdma_c = pltpu.make_async_remote_copy(
            src_ref=csend_ref, dst_ref=crecv_ref,
            send_sem=send_sems.at[2], recv_sem=recv_sems.at[2],
            device_id=nbr, device_id_type=pl.DeviceIdType.MESH)
        rdma_c.start()
        cm0 = col(we, 2 * my_x)
        cm1 = col(we, 2 * my_x + 1)

        w1cp.wait()
        w2cp.wait()
        y0, y1 = ffn(xv_ref[:, :])
        pmine_ref[:, :] = y0 * cm0 + y1 * cm1

        rdma_x.wait_recv()
        xv = xrb_ref[:, :].astype(jnp.float32)
        y0, y1 = ffn(xv)
        rdma_c.wait_recv()
        p = y0 * crecv_ref[:, 0:1] + y1 * crecv_ref[:, 1:2]
        psend_ref[:, :] = p.astype(jnp.bfloat16)
        rdma_p = pltpu.make_async_remote_copy(
            src_ref=psend_ref, dst_ref=precv_ref,
            send_sem=send_sems.at[3], recv_sem=recv_sems.at[3],
            device_id=nbr, device_id_type=pl.DeviceIdType.MESH)
        rdma_p.start()

        rdma_p.wait_recv()
        pmine_ref[:, :] = pmine_ref[:, :] + precv_ref[:, :].astype(jnp.float32)
        ocp = pltpu.make_async_copy(pmine_ref, out_hbm, csems.at[4])
        ocp.start()
        ocp.wait()

        rdma_r.wait_send()
        rdma_x.wait_send()
        rdma_c.wait_send()
        rdma_p.wait_send()

    return pl.pallas_call(
        body,
        out_shape=jax.ShapeDtypeStruct((T_SHARD, D), jnp.float32),
        in_specs=[
            pl.BlockSpec(memory_space=_HBM),
            pl.BlockSpec(memory_space=_HBM),
            pl.BlockSpec(memory_space=_HBM),
            pl.BlockSpec(memory_space=_HBM),
        ],
        out_specs=pl.BlockSpec(memory_space=_HBM),
        scratch_shapes=[
            pltpu.VMEM((T_SHARD, D), jnp.float32),
            pltpu.VMEM((D, E_LOCAL), jnp.float32),
            pltpu.VMEM((E_LOCAL, D, F), jnp.float32),
            pltpu.VMEM((E_LOCAL, F, D), jnp.float32),
            pltpu.VMEM((T_SHARD, D), jnp.bfloat16),
            pltpu.VMEM((T_SHARD, D), jnp.bfloat16),
            pltpu.VMEM((D, E_LOCAL), jnp.float32),
            pltpu.VMEM((T_SHARD, E_LOCAL), jnp.float32),
            pltpu.VMEM((T_SHARD, E_LOCAL), jnp.float32),
            pltpu.VMEM((T_SHARD, D), jnp.float32),
            pltpu.VMEM((T_SHARD, D), jnp.bfloat16),
            pltpu.VMEM((T_SHARD, D), jnp.bfloat16),
            pltpu.SemaphoreType.DMA((4,)),
            pltpu.SemaphoreType.DMA((4,)),
            pltpu.SemaphoreType.DMA((5,)),
        ],
        compiler_params=pltpu.CompilerParams(collective_id=0),
    )(*(pltpu.with_memory_space_constraint(a, _HBM)
        for a in (x, router, W1, W2)))
